# Initial kernel scaffold; baseline (speedup 1.0000x reference)
import jax
import jax.numpy as jnp
from jax.experimental import pallas as pl


def kernel(batch_fractional_bboxs, batch_obj_vecs, pos_table, word_table, W, b):
    raise NotImplementedError("write your pallas kernel here")


# speedup vs baseline: 2.6205x; 2.6205x over previous
import functools

import jax
import jax.numpy as jnp
from jax import lax
from jax.experimental import pallas as pl
from jax.experimental.pallas import tpu as pltpu
from jax.experimental.pallas import tpu_sc as plsc

IMAGE_SIZE = 300
D_POS = 64
T = 100
B = 1024
ROWS = B * T
GATHERS = ROWS * 4
BBOX_DIM = 4 * D_POS
OUT_DIM = 512

CHUNK_G = 512
GBLK = 128
NBLK = CHUNK_G // GBLK


def _sc_gather_body(frac_hbm, table_hbm, poe_hbm, frac_v, idx_v, rows_v, sem):
    nc = 2
    wid = lax.axis_index("s") * nc + lax.axis_index("c")
    g_per_w = GATHERS // 32
    g0 = wid * g_per_w

    def chunk(ci, carry):
        off = g0 + ci * CHUNK_G
        pltpu.sync_copy(frac_hbm.at[pl.ds(off, CHUNK_G)], frac_v)
        for v in range(CHUNK_G // 16):
            f = frac_v[pl.ds(v * 16, 16)]
            xi = (f * float(IMAGE_SIZE)).astype(jnp.int32)
            xi = jnp.minimum(jnp.maximum(xi, 0), IMAGE_SIZE)
            idx_v[v // 8, pl.ds((v % 8) * 16, 16)] = xi
        descs = [
            pltpu.async_copy(
                table_hbm.at[idx_v.at[j]], rows_v.at[pl.ds(j * GBLK, GBLK)], sem
            )
            for j in range(NBLK)
        ]
        for d in descs:
            d.wait()
        pltpu.sync_copy(rows_v, poe_hbm.at[pl.ds(off, CHUNK_G)])
        return carry

    lax.fori_loop(0, g_per_w // CHUNK_G, chunk, 0)


def _sc_gather(frac_flat, table):
    mesh = plsc.VectorSubcoreMesh(core_axis_name="c", subcore_axis_name="s")
    return functools.partial(
        pl.kernel,
        mesh=mesh,
        compiler_params=pltpu.CompilerParams(use_tc_tiling_on_sc=False),
        out_type=jax.ShapeDtypeStruct((GATHERS, D_POS), jnp.float32),
        scratch_types=[
            pltpu.VMEM((CHUNK_G,), jnp.float32),
            pltpu.VMEM((NBLK, GBLK), jnp.int32),
            pltpu.VMEM((CHUNK_G, D_POS), jnp.float32),
            pltpu.SemaphoreType.DMA,
        ],
    )(_sc_gather_body)(frac_flat, table)


def _base_body(wt_ref, ww_ref, b_ref, out_ref):
    out_ref[...] = (
        jnp.dot(wt_ref[...], ww_ref[...], preferred_element_type=jnp.float32)
        + b_ref[...]
    )


NB = 8


def _mm_body(poe_ref, obj_ref, w_ref, base_ref, out_ref):
    w = w_ref[...]
    base = base_ref[...]
    for i in range(NB):
        acc = jnp.dot(poe_ref[i], w, preferred_element_type=jnp.float32)
        m = (obj_ref[i] == 1).astype(jnp.float32)
        out_ref[i] = jnp.maximum(acc * m + base, 0.0)


def kernel(batch_fractional_bboxs, batch_obj_vecs, pos_table, word_table, W, b):
    frac_flat = batch_fractional_bboxs.reshape(GATHERS)
    table = jnp.pad(pos_table, ((0, 3), (0, 0)))
    Wp = W[:BBOX_DIM]
    Ww = W[BBOX_DIM:]

    base = pl.pallas_call(
        _base_body,
        out_shape=jax.ShapeDtypeStruct((T, OUT_DIM), jnp.float32),
    )(word_table, Ww, b.reshape(1, OUT_DIM))

    poe = _sc_gather(frac_flat, table)
    poe3 = poe.reshape(B, T, BBOX_DIM)

    obj3 = batch_obj_vecs.reshape(B, T, 1)
    out3 = pl.pallas_call(
        _mm_body,
        grid=(B // NB,),
        in_specs=[
            pl.BlockSpec((NB, T, BBOX_DIM), lambda i: (i, 0, 0)),
            pl.BlockSpec((NB, T, 1), lambda i: (i, 0, 0)),
            pl.BlockSpec((BBOX_DIM, OUT_DIM), lambda i: (0, 0)),
            pl.BlockSpec((T, OUT_DIM), lambda i: (0, 0)),
        ],
        out_specs=pl.BlockSpec((NB, T, OUT_DIM), lambda i: (i, 0, 0)),
        out_shape=jax.ShapeDtypeStruct((B, T, OUT_DIM), jnp.float32),
    )(poe3, obj3, Wp, base)

    return out3.reshape(ROWS, OUT_DIM)
